# Initial kernel scaffold; baseline (speedup 1.0000x reference)
import jax
import jax.numpy as jnp
from jax.experimental import pallas as pl


def kernel(sparse_indices, history, sparse_tables, item_table, W1, b1, W2, b2):
    raise NotImplementedError("write your pallas kernel here")


# speedup vs baseline: 1.1928x; 1.1928x over previous
import functools

import jax
import jax.numpy as jnp
from jax import lax
from jax.experimental import pallas as pl
from jax.experimental.pallas import tpu as pltpu
from jax.experimental.pallas import tpu_sc as plsc

F = 26
SV = 1001
E = 32
B = 4096
HIST = 200
HP = 256
DNN = 256
HID = 128
IN_DIM = F * E + E

NC, NS, L = 2, 16, 16
NW = NC * NS
BPW = B // NW
FCH = (B * F) // NW // 128


def _sc_gather_pool(tflat, fidx, item_table, histp):
    mesh = plsc.VectorSubcoreMesh(core_axis_name="c", subcore_axis_name="s")

    @functools.partial(
        pl.kernel,
        out_type=[
            jax.ShapeDtypeStruct((B * F, E), jnp.float32),
            jax.ShapeDtypeStruct((B, E), jnp.float32),
        ],
        mesh=mesh,
        compiler_params=pltpu.CompilerParams(use_tc_tiling_on_sc=False),
        scratch_types=[
            pltpu.VMEM((FCH, 128), jnp.int32),
            pltpu.VMEM((128, E), jnp.float32),
            pltpu.VMEM((BPW, 2, 128), jnp.int32),
            pltpu.VMEM((HP, E), jnp.float32),
            pltpu.VMEM((BPW, E), jnp.float32),
            pltpu.SemaphoreType.DMA,
        ],
    )
    def k(tflat_hbm, fidx_hbm, item_hbm, hidx_hbm, fout_hbm, sout_hbm,
          fidx_v, frow_v, hidx_v, hrow_v, sout_v, sem):
        wid = lax.axis_index("s") * NC + lax.axis_index("c")
        base = wid * BPW

        pltpu.sync_copy(fidx_hbm.at[wid], fidx_v)

        def fbody(j, carry):
            pltpu.async_copy(tflat_hbm.at[fidx_v.at[j]], frow_v, sem).wait()
            pltpu.sync_copy(
                frow_v, fout_hbm.at[pl.ds((wid * FCH + j) * 128, 128)])
            return carry

        lax.fori_loop(0, FCH, fbody, 0, unroll=False)

        pltpu.sync_copy(hidx_hbm.at[pl.ds(base, BPW)], hidx_v)

        def hbody(b, carry):
            cp0 = pltpu.async_copy(
                item_hbm.at[hidx_v.at[b, 0]], hrow_v.at[pl.ds(0, 128)], sem)
            cp1 = pltpu.async_copy(
                item_hbm.at[hidx_v.at[b, 1]], hrow_v.at[pl.ds(128, 128)], sem)
            cp0.wait()
            cp1.wait()

            def rbody(h, accs):
                a0, a1 = accs
                return (a0 + hrow_v[h, pl.ds(0, L)],
                        a1 + hrow_v[h, pl.ds(L, L)])

            zero = jnp.zeros((L,), jnp.float32)
            a0, a1 = lax.fori_loop(0, HIST, rbody, (zero, zero), unroll=False)
            sout_v[b, pl.ds(0, L)] = a0
            sout_v[b, pl.ds(L, L)] = a1
            return carry

        lax.fori_loop(0, BPW, hbody, 0, unroll=False)
        pltpu.sync_copy(sout_v, sout_hbm.at[pl.ds(base, BPW)])

    return k(tflat, fidx, item_table, histp)


def _mlp_body(f_ref, s_ref, h_ref, w1_ref, b1_ref, w2_ref, b2_ref, o_ref):
    x1 = f_ref[...]
    counts = jnp.sum((h_ref[...] != 0).astype(jnp.float32), axis=1,
                     keepdims=True)
    x2 = jnp.where(counts > 0.0, s_ref[...] / jnp.maximum(counts, 1.0), 0.0)
    h = jnp.dot(x1, w1_ref[0:F * E, :], preferred_element_type=jnp.float32,
                precision=lax.Precision.HIGHEST)
    h = h + jnp.dot(x2, w1_ref[F * E:IN_DIM, :],
                    preferred_element_type=jnp.float32,
                    precision=lax.Precision.HIGHEST)
    h = jnp.maximum(h + b1_ref[...], 0.0)
    o_ref[...] = jnp.dot(h, w2_ref[...], preferred_element_type=jnp.float32,
                         precision=lax.Precision.HIGHEST) + b2_ref[...]


def _mlp(femb, seq_sum, history, W1, b1, W2, b2):
    BM = 256
    return pl.pallas_call(
        _mlp_body,
        grid=(B // BM,),
        in_specs=[
            pl.BlockSpec((BM, F * E), lambda i: (i, 0)),
            pl.BlockSpec((BM, E), lambda i: (i, 0)),
            pl.BlockSpec((BM, HIST), lambda i: (i, 0)),
            pl.BlockSpec((IN_DIM, DNN), lambda i: (0, 0)),
            pl.BlockSpec((1, DNN), lambda i: (0, 0)),
            pl.BlockSpec((DNN, HID), lambda i: (0, 0)),
            pl.BlockSpec((1, HID), lambda i: (0, 0)),
        ],
        out_specs=pl.BlockSpec((BM, HID), lambda i: (i, 0)),
        out_shape=jax.ShapeDtypeStruct((B, HID), jnp.float32),
    )(femb, seq_sum, history, W1, b1[None, :], W2, b2[None, :])


def kernel(sparse_indices, history, sparse_tables, item_table, W1, b1, W2, b2):
    tflat = sparse_tables.reshape(F * SV, E)
    fidx = (sparse_indices.astype(jnp.int32)
            + (jnp.arange(F, dtype=jnp.int32) * SV)[None, :])
    fidx = fidx.reshape(NW, FCH, 128)
    histp = jnp.pad(history.astype(jnp.int32), ((0, 0), (0, HP - HIST)))
    histp = histp.reshape(B, 2, 128)
    frows, seq_sum = _sc_gather_pool(tflat, fidx, item_table, histp)
    femb = frows.reshape(B, F * E)
    return _mlp(femb, seq_sum, history, W1, b1, W2, b2)
